# baseline (speedup 1.0000x reference)
import functools
import math

import jax
import jax.numpy as jnp
from jax import lax
from jax.experimental import pallas as pl
from jax.experimental.pallas import tpu as pltpu
from jax.experimental.pallas import tpu_sc as plsc

_BATCH = 1024
_SIZE = 100000
_SMOOTHING = 0.1
_CONFIDENCE = 1.0 - _SMOOTHING
_FILL = _SMOOTHING / (_SIZE - 2)
_DELTA = _CONFIDENCE - _FILL
_CONST = _BATCH * ((_SIZE - 1) * _FILL * math.log(_FILL)
                   + _CONFIDENCE * math.log(_CONFIDENCE))

_NC, _NS, _L = 2, 16, 16
_NW = _NC * _NS
_BPW = _BATCH // _NW

_sc_mesh = plsc.VectorSubcoreMesh(core_axis_name="c", subcore_axis_name="s")


@functools.partial(
    pl.kernel,
    out_type=jax.ShapeDtypeStruct((_NW * _L,), jnp.float32),
    mesh=_sc_mesh,
    compiler_params=pltpu.CompilerParams(needs_layout_passes=False),
    scratch_types=[
        pltpu.VMEM((_BPW,), jnp.int32),
        pltpu.VMEM((_BPW, 8, 128), jnp.float32),
        pltpu.VMEM((_L,), jnp.float32),
        pltpu.SemaphoreType.DMA,
    ],
)
def _sc_gather(x_hbm, tgt_hbm, out_hbm, tgt_v, win_v, acc_v, sem):
    wid = lax.axis_index("s") * _NC + lax.axis_index("c")
    base = wid * _BPW
    pltpu.sync_copy(tgt_hbm.at[pl.ds(base, _BPW)], tgt_v)
    lanes = lax.broadcasted_iota(jnp.int32, (_L,), 0)

    ts = []
    copies = []
    for c in range(_BPW // _L):
        tvec = tgt_v[pl.ds(c * _L, _L)]
        for l in range(_L):
            k = c * _L + l
            t = jnp.sum(jnp.where(lanes == l, tvec, 0))
            ts.append(t)
            start128 = pl.multiple_of(t & (-128), 128)
            row8 = pl.multiple_of(base + (k & ~7), 8)
            copies.append(pltpu.async_copy(
                x_hbm.at[pl.ds(row8, 8), pl.ds(start128, 128)],
                win_v.at[k], sem))
    for cp in copies:
        cp.wait()

    acc = jnp.zeros((_L,), jnp.float32)
    for k in range(_BPW):
        t = ts[k]
        sub = (t & 127) & (-_L)
        w = win_v[k, k & 7, pl.ds(sub, _L)]
        acc = acc + jnp.where(lanes == (t & (_L - 1)), w, 0.0)
    acc_v[...] = acc
    pltpu.sync_copy(acc_v, out_hbm.at[pl.ds(wid * _L, _L)])


_W = 2048
_GRID = (_SIZE + _W - 1) // _W


def _sum_body(g_ref, x_ref, out_ref, acc_ref):
    j = pl.program_id(0)

    @pl.when(j == 0)
    def _init():
        acc_ref[0] = 0.0

    @pl.when(j < _GRID - 1)
    def _full():
        acc_ref[0] += jnp.sum(x_ref[...])

    @pl.when(j == _GRID - 1)
    def _last():
        ids = lax.broadcasted_iota(jnp.int32, (_BATCH, _W), 1) + j * _W
        s1 = acc_ref[0] + jnp.sum(jnp.where(ids < _SIZE, x_ref[...], 0.0))
        s2 = jnp.sum(g_ref[...])
        loss = _CONST - _FILL * s1 - _DELTA * s2
        out_ref[0, 0] = loss.astype(jnp.float32)


@jax.jit
def kernel(x, target):
    g = _sc_gather(x, target)
    out = pl.pallas_call(
        _sum_body,
        grid=(_GRID,),
        in_specs=[
            pl.BlockSpec((_NW * _L // 128, 128), lambda j: (0, 0)),
            pl.BlockSpec((_BATCH, _W), lambda j: (0, j)),
        ],
        out_specs=pl.BlockSpec(memory_space=pltpu.SMEM),
        out_shape=jax.ShapeDtypeStruct((1, 1), jnp.float32),
        scratch_shapes=[pltpu.SMEM((1,), jnp.float32)],
    )(g.reshape(_NW * _L // 128, 128), x)
    return out[0, 0]


# speedup vs baseline: 1.0503x; 1.0503x over previous
import functools
import math

import jax
import jax.numpy as jnp
from jax import lax
from jax.experimental import pallas as pl
from jax.experimental.pallas import tpu as pltpu
from jax.experimental.pallas import tpu_sc as plsc

_BATCH = 1024
_SIZE = 100000
_SMOOTHING = 0.1
_CONFIDENCE = 1.0 - _SMOOTHING
_FILL = _SMOOTHING / (_SIZE - 2)
_DELTA = _CONFIDENCE - _FILL
_CONST = _BATCH * ((_SIZE - 1) * _FILL * math.log(_FILL)
                   + _CONFIDENCE * math.log(_CONFIDENCE))

_NC, _NS, _L = 2, 16, 16
_NW = _NC * _NS
_BPW = _BATCH // _NW

_sc_mesh = plsc.VectorSubcoreMesh(core_axis_name="c", subcore_axis_name="s")


@functools.partial(
    pl.kernel,
    out_type=jax.ShapeDtypeStruct((_NW * _L,), jnp.float32),
    mesh=_sc_mesh,
    compiler_params=pltpu.CompilerParams(needs_layout_passes=False),
    scratch_types=[
        pltpu.VMEM((_BPW,), jnp.int32),
        pltpu.VMEM((_BPW, 8, 128), jnp.float32),
        pltpu.VMEM((_L,), jnp.float32),
        pltpu.SemaphoreType.DMA,
    ],
)
def _sc_gather(x_hbm, tgt_hbm, out_hbm, tgt_v, win_v, acc_v, sem):
    wid = lax.axis_index("s") * _NC + lax.axis_index("c")
    base = wid * _BPW
    pltpu.sync_copy(tgt_hbm.at[pl.ds(base, _BPW)], tgt_v)
    lanes = lax.broadcasted_iota(jnp.int32, (_L,), 0)

    ts = []
    copies = []
    for c in range(_BPW // _L):
        tvec = tgt_v[pl.ds(c * _L, _L)]
        for l in range(_L):
            k = c * _L + l
            t = jnp.sum(jnp.where(lanes == l, tvec, 0))
            ts.append(t)
            start128 = pl.multiple_of(t & (-128), 128)
            row8 = pl.multiple_of(base + (k & ~7), 8)
            copies.append(pltpu.async_copy(
                x_hbm.at[pl.ds(row8, 8), pl.ds(start128, 128)],
                win_v.at[k], sem))
    for cp in copies:
        cp.wait()

    acc = jnp.zeros((_L,), jnp.float32)
    for k in range(_BPW):
        t = ts[k]
        sub = (t & 127) & (-_L)
        w = win_v[k, k & 7, pl.ds(sub, _L)]
        acc = acc + jnp.where(lanes == (t & (_L - 1)), w, 0.0)
    acc_v[...] = acc
    pltpu.sync_copy(acc_v, out_hbm.at[pl.ds(wid * _L, _L)])


_W = 2048
_GRID = (_SIZE + _W - 1) // _W
_NSPLIT = 4
_RS = _BATCH // _NSPLIT


def _sum_body(g_ref, *refs):
    x_refs = refs[:_NSPLIT]
    out_ref = refs[_NSPLIT]
    acc_ref = refs[_NSPLIT + 1]
    j = pl.program_id(0)

    @pl.when(j == 0)
    def _init():
        acc_ref[0] = 0.0

    @pl.when(j < _GRID - 1)
    def _full():
        s = x_refs[0][...]
        for r in x_refs[1:]:
            s = s + r[...]
        acc_ref[0] += jnp.sum(s)

    @pl.when(j == _GRID - 1)
    def _last():
        ids = lax.broadcasted_iota(jnp.int32, (_RS, _W), 1) + j * _W
        m = ids < _SIZE
        s = jnp.where(m, x_refs[0][...], 0.0)
        for r in x_refs[1:]:
            s = s + jnp.where(m, r[...], 0.0)
        s1 = acc_ref[0] + jnp.sum(s)
        s2 = jnp.sum(g_ref[...])
        loss = _CONST - _FILL * s1 - _DELTA * s2
        out_ref[0, 0] = loss.astype(jnp.float32)


def _mk_spec(i):
    return pl.BlockSpec((_RS, _W), lambda j, i=i: (i, j))


@jax.jit
def kernel(x, target):
    g = _sc_gather(x, target)
    out = pl.pallas_call(
        _sum_body,
        grid=(_GRID,),
        in_specs=[pl.BlockSpec((_NW * _L // 128, 128), lambda j: (0, 0))]
        + [_mk_spec(i) for i in range(_NSPLIT)],
        out_specs=pl.BlockSpec(memory_space=pltpu.SMEM),
        out_shape=jax.ShapeDtypeStruct((1, 1), jnp.float32),
        scratch_shapes=[pltpu.SMEM((1,), jnp.float32)],
    )(g.reshape(_NW * _L // 128, 128), *([x] * _NSPLIT))
    return out[0, 0]
